# baseline (speedup 1.0000x reference)
import jax
import jax.numpy as jnp
from jax.experimental import pallas as pl

N_USERS = 4096
NUM_SUBREDDITS = 20000
X_DIM = 64
SUB_REP_DIM = 3

BM = 128


def _agg_kernel(x_ref, s_ref, r_ref, o_ref):
    s = s_ref[...].astype(jnp.bfloat16)
    r = r_ref[...].astype(jnp.bfloat16)
    acc = jnp.dot(s, r, preferred_element_type=jnp.float32)
    o_ref[:, :X_DIM] = x_ref[...]
    o_ref[:, X_DIM:] = acc


def kernel(x, S, R):
    grid = (N_USERS // BM,)
    out = pl.pallas_call(
        _agg_kernel,
        grid=grid,
        in_specs=[
            pl.BlockSpec((BM, X_DIM), lambda i: (i, 0)),
            pl.BlockSpec((BM, NUM_SUBREDDITS), lambda i: (i, 0)),
            pl.BlockSpec((NUM_SUBREDDITS, SUB_REP_DIM), lambda i: (0, 0)),
        ],
        out_specs=pl.BlockSpec((BM, X_DIM + SUB_REP_DIM), lambda i: (i, 0)),
        out_shape=jax.ShapeDtypeStruct((N_USERS, X_DIM + SUB_REP_DIM), jnp.float32),
    )(x, S, R)
    return out
